# Initial kernel scaffold; baseline (speedup 1.0000x reference)
import jax
import jax.numpy as jnp
from jax.experimental import pallas as pl


def kernel(encoder_output, Wq, bq, Wk, bk, Wv, bv, Wo, bo, W1, b1, W2, b2, g1, be1, g2, be2, Wout, bout):
    raise NotImplementedError("write your pallas kernel here")


# speedup vs baseline: 2.7289x; 2.7289x over previous
import math

import jax
import jax.numpy as jnp
import numpy as np
from jax.experimental import pallas as pl

D_MODEL = 768
N_HEADS = 12
DH = D_MODEL // N_HEADS
D_FF = 3072
HORIZON = 22
HQ = 24
B_SZ = 4
S_LEN = 8192
NQ = N_HEADS * HQ
U_TOP = min(HORIZON, 5 * int(math.ceil(math.log(HORIZON))))
S_BLK = 2048
NS = S_LEN // S_BLK


def _make_pe_pad():
    pos = np.arange(HORIZON, dtype=np.float32)[:, None]
    div = np.exp(np.arange(0, D_MODEL, 2, dtype=np.float32)
                 * (-math.log(10000.0) / D_MODEL))
    pe = np.zeros((HQ, D_MODEL), dtype=np.float32)
    pe[:HORIZON, 0::2] = np.sin(pos * div)
    pe[:HORIZON, 1::2] = np.cos(pos * div)
    return pe


def _hq_index_rows():
    hq = np.arange(NQ)
    h = (hq // HQ).astype(np.float32)[None, :]
    q = (hq % HQ).astype(np.float32)[None, :]
    return h, q


def _eye(n):
    ii = jax.lax.broadcasted_iota(jnp.int32, (n, n), 0)
    jj = jax.lax.broadcasted_iota(jnp.int32, (n, n), 1)
    return (ii == jj).astype(jnp.float32)


def _to_col(row):
    n = row.shape[1]
    return jax.lax.dot_general(_eye(n), row, (((1,), (1,)), ((), ())),
                               precision=jax.lax.Precision.HIGHEST,
                               preferred_element_type=jnp.float32)


def _prep_body(pe_ref, wq_ref, bq_ref, wk_ref, a_ref):
    del wk_ref
    qp = jax.lax.dot_general(pe_ref[...], wq_ref[...],
                             (((1,), (1,)), ((), ())),
                             preferred_element_type=jnp.float32) + bq_ref[...]
    lanes = jax.lax.broadcasted_iota(jnp.int32, (1, D_MODEL), 1)
    blocks = []
    for h in range(N_HEADS):
        mask = ((lanes >= h * DH) & (lanes < (h + 1) * DH)).astype(jnp.float32)
        blocks.append(qp * mask)
    a_ref[...] = jnp.concatenate(blocks, axis=0)


def _main_body(enc_ref, wk_ref, a_ref, wv_ref, bv_ref, m_ref, l_ref,
               ssum_ref, vsum_ref, pacc_ref):
    s = pl.program_id(1)
    enc = enc_ref[0]
    k_blk = jax.lax.dot_general(enc, wk_ref[...], (((1,), (1,)), ((), ())),
                                preferred_element_type=jnp.float32)
    scores = jax.lax.dot_general(k_blk, a_ref[...], (((1,), (1,)), ((), ())),
                                 preferred_element_type=jnp.float32)
    v_blk = jax.lax.dot_general(enc, wv_ref[...], (((1,), (1,)), ((), ())),
                                preferred_element_type=jnp.float32) + bv_ref[...]
    blk_max = jnp.max(scores, axis=0, keepdims=True)
    blk_sum = jnp.sum(scores, axis=0, keepdims=True)
    vsum = jnp.sum(v_blk, axis=0, keepdims=True)
    scale = 1.0 / math.sqrt(DH)

    @pl.when(s == 0)
    def _init():
        p = jnp.exp((scores - blk_max) * scale)
        m_ref[0] = blk_max
        l_ref[0] = jnp.sum(p, axis=0, keepdims=True)
        ssum_ref[0] = blk_sum
        vsum_ref[0] = vsum
        pacc_ref[0] = jax.lax.dot_general(p, v_blk, (((0,), (0,)), ((), ())),
                                          preferred_element_type=jnp.float32)

    @pl.when(s != 0)
    def _update():
        m_old = m_ref[0]
        m_new = jnp.maximum(m_old, blk_max)
        alpha = jnp.exp((m_old - m_new) * scale)
        p = jnp.exp((scores - m_new) * scale)
        m_ref[0] = m_new
        l_ref[0] = l_ref[0] * alpha + jnp.sum(p, axis=0, keepdims=True)
        ssum_ref[0] = ssum_ref[0] + blk_sum
        vsum_ref[0] = vsum_ref[0] + vsum
        pacc_ref[0] = (pacc_ref[0] * _to_col(alpha)
                       + jax.lax.dot_general(p, v_blk, (((0,), (0,)), ((), ())),
                                             preferred_element_type=jnp.float32))


def _layer_norm(x, g, b, eps=1e-5):
    m = jnp.mean(x, axis=-1, keepdims=True)
    v = jnp.mean((x - m) ** 2, axis=-1, keepdims=True)
    return (x - m) / jnp.sqrt(v + eps) * g + b


def _tail_body(hrow_ref, qrow_ref, m_ref, l_ref, ssum_ref, vsum_ref, pacc_ref,
               wo_ref, bo_ref, w1_ref, b1_ref, w2_ref, b2_ref,
               g1_ref, be1_ref, g2_ref, be2_ref, wout_ref,
               fc_ref, y_ref):
    hrow = hrow_ref[...]
    qrow = qrow_ref[...]
    hcol = _to_col(hrow)
    qcol = _to_col(qrow)

    lazy_cols, l_cols = [], []
    for b in range(B_SZ):
        mstat = m_ref[b] - ssum_ref[b] / S_LEN
        mcol = _to_col(mstat)
        better = (mstat > mcol) | ((mstat == mcol) & (qrow < qcol))
        valid = (hrow == hcol) & (qrow < float(HORIZON))
        rank = jnp.sum((better & valid).astype(jnp.float32),
                       axis=1, keepdims=True)
        lazy_cols.append((rank >= float(U_TOP)).astype(jnp.float32)[None])
        l_cols.append(_to_col(l_ref[b])[None])
    lazy3 = jnp.concatenate(lazy_cols, axis=0)
    l3 = jnp.concatenate(l_cols, axis=0)

    ctx_all = pacc_ref[...] / l3

    vmean3 = jnp.concatenate([vsum_ref[b][None] for b in range(B_SZ)],
                             axis=0) / S_LEN

    lanes = jax.lax.broadcasted_iota(jnp.int32, (1, 1, D_MODEL), 2)
    attn = jnp.zeros((B_SZ, HQ, D_MODEL), jnp.float32)
    for h in range(N_HEADS):
        mask = ((lanes >= h * DH) & (lanes < (h + 1) * DH)).astype(jnp.float32)
        act = ctx_all[:, h * HQ:(h + 1) * HQ, :]
        lz = lazy3[:, h * HQ:(h + 1) * HQ, :]
        sel = act * (1.0 - lz) + vmean3 * lz
        attn = attn + sel * mask
    attn = attn.reshape(B_SZ * HQ, D_MODEL)

    proj = jax.lax.dot_general(attn, wo_ref[...], (((1,), (1,)), ((), ())),
                               preferred_element_type=jnp.float32) + bo_ref[...]
    x = _layer_norm(proj + proj, g1_ref[...], be1_ref[...])
    h1 = jax.lax.dot_general(x, w1_ref[...], (((1,), (1,)), ((), ())),
                             preferred_element_type=jnp.float32) + b1_ref[...]
    h1 = jnp.maximum(h1, 0.0)
    ff = jax.lax.dot_general(h1, w2_ref[...], (((1,), (1,)), ((), ())),
                             preferred_element_type=jnp.float32) + b2_ref[...]
    y = _layer_norm(x + ff, g2_ref[...], be2_ref[...])
    fc = jnp.sum(y * wout_ref[...], axis=1, keepdims=True)
    y_ref[...] = y
    fc_ref[...] = fc


def kernel(encoder_output, Wq, bq, Wk, bk, Wv, bv, Wo, bo,
           W1, b1, W2, b2, g1, be1, g2, be2, Wout, bout):
    pe_pad = jnp.asarray(_make_pe_pad())
    hrow_np, qrow_np = _hq_index_rows()

    a_mat = pl.pallas_call(
        _prep_body,
        out_shape=jax.ShapeDtypeStruct((NQ, D_MODEL), jnp.float32),
    )(pe_pad, Wq, bq, Wk)

    stats_shape = jax.ShapeDtypeStruct((B_SZ, 1, NQ), jnp.float32)
    m_s, l_s, ssum_s, vsum_s, pacc_s = pl.pallas_call(
        _main_body,
        grid=(B_SZ, NS),
        in_specs=[
            pl.BlockSpec((1, S_BLK, D_MODEL), lambda b, s: (b, s, 0)),
            pl.BlockSpec((D_MODEL, D_MODEL), lambda b, s: (0, 0)),
            pl.BlockSpec((NQ, D_MODEL), lambda b, s: (0, 0)),
            pl.BlockSpec((D_MODEL, D_MODEL), lambda b, s: (0, 0)),
            pl.BlockSpec((D_MODEL,), lambda b, s: (0,)),
        ],
        out_specs=[
            pl.BlockSpec((1, 1, NQ), lambda b, s: (b, 0, 0)),
            pl.BlockSpec((1, 1, NQ), lambda b, s: (b, 0, 0)),
            pl.BlockSpec((1, 1, NQ), lambda b, s: (b, 0, 0)),
            pl.BlockSpec((1, 1, D_MODEL), lambda b, s: (b, 0, 0)),
            pl.BlockSpec((1, NQ, D_MODEL), lambda b, s: (b, 0, 0)),
        ],
        out_shape=[
            stats_shape, stats_shape, stats_shape,
            jax.ShapeDtypeStruct((B_SZ, 1, D_MODEL), jnp.float32),
            jax.ShapeDtypeStruct((B_SZ, NQ, D_MODEL), jnp.float32),
        ],
    )(encoder_output, Wk, a_mat, Wv, bv)

    fc_pad, y_pad = pl.pallas_call(
        _tail_body,
        out_shape=[
            jax.ShapeDtypeStruct((B_SZ * HQ, 1), jnp.float32),
            jax.ShapeDtypeStruct((B_SZ * HQ, D_MODEL), jnp.float32),
        ],
    )(jnp.asarray(hrow_np), jnp.asarray(qrow_np),
      m_s, l_s, ssum_s, vsum_s, pacc_s,
      Wo, bo, W1, b1, W2, b2, g1, be1, g2, be2, Wout)

    y = y_pad.reshape(B_SZ, HQ, D_MODEL)[:, :HORIZON, :]
    forecasts = fc_pad.reshape(B_SZ, HQ)[:, :HORIZON] + bout
    return forecasts, y
